# baseline (speedup 1.0000x reference)
import jax
import jax.numpy as jnp
from jax.experimental import pallas as pl
from jax.experimental.pallas import tpu as pltpu

B, S, HID = 2, 2048, 2048
H, KVH, D = 16, 4, 128
THETA = 10000.0
GROUPS = H // KVH

BM = 512
BQ = 512
BK = 512
MQ = GROUPS * BQ
SCALE = 1.4426950408889634 / (D ** 0.5)


def _rope(x, cos, sin):
    rot = jnp.concatenate([-x[:, D // 2:], x[:, : D // 2]], axis=1)
    return x * cos + rot * sin


def _qkv_proj_kernel(x_ref, w_ref, cos_ref, sin_ref, q_ref, k_ref, v_ref):
    x = x_ref[0]
    cos = cos_ref[...]
    sin = sin_ref[...]
    for c in range((H + 2 * KVH) // 4):
        y = jnp.dot(x, w_ref[:, 4 * c * D:4 * (c + 1) * D],
                    preferred_element_type=jnp.float32)
        for j in range(4):
            h = 4 * c + j
            yh = y[:, j * D:(j + 1) * D]
            if h < H:
                q_ref[0, h, :, :] = _rope(yh, cos, sin).astype(jnp.bfloat16)
            elif h < H + KVH:
                k_ref[0, h - H, :, :] = _rope(yh, cos, sin).astype(jnp.bfloat16)
            else:
                v_ref[0, h - H - KVH, :, :] = yh.astype(jnp.bfloat16)


def _flash_kernel(q_ref, k_ref, v_ref, wo_ref, mask_ref, o_ref):
    qi = pl.program_id(1)

    m0 = jnp.full((MQ, 1), -1e30, jnp.bfloat16)
    l0 = jnp.zeros((MQ, 1), jnp.float32)
    acc0 = jnp.zeros((MQ, D), jnp.float32)

    out = jnp.zeros((BQ, HID), jnp.float32)
    for g in range(KVH):
        q = q_ref[0, GROUPS * g:GROUPS * (g + 1)].reshape(MQ, D)

        def block(kb, carry, masked):
            m, l, acc = carry
            ks = k_ref[0, g, pl.ds(kb * BK, BK), :]
            vs = v_ref[0, g, pl.ds(kb * BK, BK), :]
            s = jax.lax.dot_general(q, ks, (((1,), (1,)), ((), ())),
                                    preferred_element_type=jnp.float32
                                    ).astype(jnp.bfloat16)
            if masked:
                s = s + mask_ref[...]
            m_new = jnp.maximum(m, s.max(axis=1, keepdims=True))
            alpha = jax.lax.exp2((m - m_new).astype(jnp.float32))
            p = jax.lax.exp2(s - m_new)
            l_new = l * alpha + p.sum(axis=1, keepdims=True, dtype=jnp.float32)
            acc_new = acc * alpha + jnp.dot(p, vs,
                                            preferred_element_type=jnp.float32)
            return m_new, l_new, acc_new

        carry = jax.lax.fori_loop(0, qi, lambda kb, c: block(kb, c, False),
                                  (m0, l0, acc0))
        m, l, acc = block(qi, carry, True)

        attn = (acc / l).astype(jnp.bfloat16)
        attn_w = jnp.concatenate(
            [attn[j * BQ:(j + 1) * BQ, :] for j in range(GROUPS)], axis=1)
        out = out + jnp.dot(attn_w, wo_ref[GROUPS * D * g:GROUPS * D * (g + 1), :],
                            preferred_element_type=jnp.float32)

    o_ref[0] = out


def kernel(hidden_states, Wq, Wk, Wv, Wo):
    inv_freq = 1.0 / (THETA ** (jnp.arange(0, D, 2, dtype=jnp.float32) / D))
    t = jnp.arange(S, dtype=jnp.float32)
    freqs = jnp.outer(t, inv_freq)
    emb = jnp.concatenate([freqs, freqs], axis=-1)
    cos = jnp.cos(emb)
    sin = jnp.sin(emb)
    wqkv = jnp.concatenate([Wq * SCALE, Wk, Wv], axis=1).astype(jnp.bfloat16)
    r = jnp.arange(MQ, dtype=jnp.int32) % BQ
    c = jnp.arange(BK, dtype=jnp.int32)
    mask_add = jnp.where(r[:, None] >= c[None, :], 0.0, -1e30
                         ).astype(jnp.bfloat16)
    wo16 = Wo.astype(jnp.bfloat16)
    x16 = hidden_states.astype(jnp.bfloat16)

    q, k, v = pl.pallas_call(
        _qkv_proj_kernel,
        grid=(B, S // BM),
        in_specs=[
            pl.BlockSpec((1, BM, HID), lambda b, m: (b, m, 0)),
            pl.BlockSpec((HID, (H + 2 * KVH) * D), lambda b, m: (0, 0)),
            pl.BlockSpec((BM, D), lambda b, m: (m, 0)),
            pl.BlockSpec((BM, D), lambda b, m: (m, 0)),
        ],
        out_specs=[
            pl.BlockSpec((1, H, BM, D), lambda b, m: (b, 0, m, 0)),
            pl.BlockSpec((1, KVH, BM, D), lambda b, m: (b, 0, m, 0)),
            pl.BlockSpec((1, KVH, BM, D), lambda b, m: (b, 0, m, 0)),
        ],
        out_shape=[
            jax.ShapeDtypeStruct((B, H, S, D), jnp.bfloat16),
            jax.ShapeDtypeStruct((B, KVH, S, D), jnp.bfloat16),
            jax.ShapeDtypeStruct((B, KVH, S, D), jnp.bfloat16),
        ],
        compiler_params=pltpu.CompilerParams(
            dimension_semantics=("parallel", "arbitrary")),
    )(x16, wqkv, cos, sin)

    out = pl.pallas_call(
        _flash_kernel,
        grid=(B, S // BQ),
        in_specs=[
            pl.BlockSpec((1, H, BQ, D), lambda b, i: (b, 0, i, 0)),
            pl.BlockSpec((1, KVH, S, D), lambda b, i: (b, 0, 0, 0)),
            pl.BlockSpec((1, KVH, S, D), lambda b, i: (b, 0, 0, 0)),
            pl.BlockSpec((H * D, HID), lambda b, i: (0, 0)),
            pl.BlockSpec((MQ, BK), lambda b, i: (0, 0)),
        ],
        out_specs=pl.BlockSpec((1, BQ, HID), lambda b, i: (b, i, 0)),
        out_shape=jax.ShapeDtypeStruct((B, S, HID), jnp.float32),
        compiler_params=pltpu.CompilerParams(
            dimension_semantics=("parallel", "arbitrary")),
    )(q, k, v, wo16, mask_add)

    return out


# speedup vs baseline: 1.0005x; 1.0005x over previous
import jax
import jax.numpy as jnp
from jax.experimental import pallas as pl
from jax.experimental.pallas import tpu as pltpu

B, S, HID = 2, 2048, 2048
H, KVH, D = 16, 4, 128
THETA = 10000.0
GROUPS = H // KVH

BM = 512
BQ = 512
BK = 512
MQ = GROUPS * BQ
SCALE = 1.4426950408889634 / (D ** 0.5)


def _rope(x, cos, sin):
    rot = jnp.concatenate([-x[:, D // 2:], x[:, : D // 2]], axis=1)
    return x * cos + rot * sin


def _qkv_proj_kernel(x_ref, w_ref, cos_ref, sin_ref, q_ref, k_ref, v_ref):
    x = x_ref[0]
    cos = cos_ref[...]
    sin = sin_ref[...]
    for c in range((H + 2 * KVH) // 4):
        y = jnp.dot(x, w_ref[:, 4 * c * D:4 * (c + 1) * D],
                    preferred_element_type=jnp.float32)
        for j in range(4):
            h = 4 * c + j
            yh = y[:, j * D:(j + 1) * D]
            if h < H:
                q_ref[0, h, :, :] = _rope(yh, cos, sin).astype(jnp.bfloat16)
            elif h < H + KVH:
                k_ref[0, h - H, :, :] = _rope(yh, cos, sin).astype(jnp.bfloat16)
            else:
                v_ref[0, h - H - KVH, :, :] = yh.astype(jnp.bfloat16)


def _flash_kernel(q_ref, k_ref, v_ref, wo_ref, mask_ref, o_ref):
    qi = pl.program_id(1)

    m0 = jnp.full((MQ, 1), -1e30, jnp.bfloat16)
    l0 = jnp.zeros((MQ, 1), jnp.float32)
    acc0 = jnp.zeros((MQ, D), jnp.float32)

    out = jnp.zeros((BQ, HID), jnp.float32)
    for g in range(KVH):
        q = q_ref[0, GROUPS * g:GROUPS * (g + 1)].reshape(MQ, D)

        def block(kb, carry, masked):
            m, l, acc = carry
            ks = k_ref[0, g, pl.ds(kb * BK, BK), :]
            vs = v_ref[0, g, pl.ds(kb * BK, BK), :]
            s = jax.lax.dot_general(q, ks, (((1,), (1,)), ((), ())),
                                    preferred_element_type=jnp.float32
                                    ).astype(jnp.bfloat16)
            if masked:
                s = s + mask_ref[...]
            m_new = jnp.maximum(m, s.max(axis=1, keepdims=True))
            alpha = jax.lax.exp2((m - m_new).astype(jnp.float32))
            p = jax.lax.exp2(s - m_new)
            l_new = l * alpha + p.sum(axis=1, keepdims=True, dtype=jnp.float32)
            acc_new = acc * alpha + jnp.dot(p, vs,
                                            preferred_element_type=jnp.float32)
            return m_new, l_new, acc_new

        carry = jax.lax.fori_loop(0, qi, lambda kb, c: block(kb, c, False),
                                  (m0, l0, acc0))
        m, l, acc = block(qi, carry, True)

        attn = (acc * (1.0 / l)).astype(jnp.bfloat16)
        attn_w = jnp.concatenate(
            [attn[j * BQ:(j + 1) * BQ, :] for j in range(GROUPS)], axis=1)
        out = out + jnp.dot(attn_w, wo_ref[GROUPS * D * g:GROUPS * D * (g + 1), :],
                            preferred_element_type=jnp.float32)

    o_ref[0] = out


def kernel(hidden_states, Wq, Wk, Wv, Wo):
    inv_freq = 1.0 / (THETA ** (jnp.arange(0, D, 2, dtype=jnp.float32) / D))
    t = jnp.arange(S, dtype=jnp.float32)
    freqs = jnp.outer(t, inv_freq)
    emb = jnp.concatenate([freqs, freqs], axis=-1)
    cos = jnp.cos(emb)
    sin = jnp.sin(emb)
    wqkv = jnp.concatenate([Wq * SCALE, Wk, Wv], axis=1).astype(jnp.bfloat16)
    r = jnp.arange(MQ, dtype=jnp.int32) % BQ
    c = jnp.arange(BK, dtype=jnp.int32)
    mask_add = jnp.where(r[:, None] >= c[None, :], 0.0, -1e30
                         ).astype(jnp.bfloat16)
    wo16 = Wo.astype(jnp.bfloat16)
    x16 = hidden_states.astype(jnp.bfloat16)

    q, k, v = pl.pallas_call(
        _qkv_proj_kernel,
        grid=(B, S // BM),
        in_specs=[
            pl.BlockSpec((1, BM, HID), lambda b, m: (b, m, 0)),
            pl.BlockSpec((HID, (H + 2 * KVH) * D), lambda b, m: (0, 0)),
            pl.BlockSpec((BM, D), lambda b, m: (m, 0)),
            pl.BlockSpec((BM, D), lambda b, m: (m, 0)),
        ],
        out_specs=[
            pl.BlockSpec((1, H, BM, D), lambda b, m: (b, 0, m, 0)),
            pl.BlockSpec((1, KVH, BM, D), lambda b, m: (b, 0, m, 0)),
            pl.BlockSpec((1, KVH, BM, D), lambda b, m: (b, 0, m, 0)),
        ],
        out_shape=[
            jax.ShapeDtypeStruct((B, H, S, D), jnp.bfloat16),
            jax.ShapeDtypeStruct((B, KVH, S, D), jnp.bfloat16),
            jax.ShapeDtypeStruct((B, KVH, S, D), jnp.bfloat16),
        ],
        compiler_params=pltpu.CompilerParams(
            dimension_semantics=("parallel", "arbitrary")),
    )(x16, wqkv, cos, sin)

    out = pl.pallas_call(
        _flash_kernel,
        grid=(B, S // BQ),
        in_specs=[
            pl.BlockSpec((1, H, BQ, D), lambda b, i: (b, 0, i, 0)),
            pl.BlockSpec((1, KVH, S, D), lambda b, i: (b, 0, 0, 0)),
            pl.BlockSpec((1, KVH, S, D), lambda b, i: (b, 0, 0, 0)),
            pl.BlockSpec((H * D, HID), lambda b, i: (0, 0)),
            pl.BlockSpec((MQ, BK), lambda b, i: (0, 0)),
        ],
        out_specs=pl.BlockSpec((1, BQ, HID), lambda b, i: (b, i, 0)),
        out_shape=jax.ShapeDtypeStruct((B, S, HID), jnp.float32),
        compiler_params=pltpu.CompilerParams(
            dimension_semantics=("parallel", "arbitrary")),
    )(q, k, v, wo16, mask_add)

    return out


# speedup vs baseline: 1.0172x; 1.0167x over previous
import jax
import jax.numpy as jnp
from jax.experimental import pallas as pl
from jax.experimental.pallas import tpu as pltpu

B, S, HID = 2, 2048, 2048
H, KVH, D = 16, 4, 128
THETA = 10000.0
GROUPS = H // KVH

BM = 512
BQ = 512
BK = 512
MQ = GROUPS * BQ
SCALE = 1.4426950408889634 / (D ** 0.5)


def _rope(x, cos, sin):
    rot = jnp.concatenate([-x[:, D // 2:], x[:, : D // 2]], axis=1)
    return x * cos + rot * sin


def _qkv_proj_kernel(x_ref, w_ref, cos_ref, sin_ref, q_ref, k_ref, v_ref):
    x = x_ref[0]
    cos = cos_ref[...]
    sin = sin_ref[...]
    for c in range((H + 2 * KVH) // 4):
        y = jnp.dot(x, w_ref[:, 4 * c * D:4 * (c + 1) * D],
                    preferred_element_type=jnp.float32)
        for j in range(4):
            h = 4 * c + j
            yh = y[:, j * D:(j + 1) * D]
            if h < H:
                q_ref[0, h, :, :] = _rope(yh, cos, sin).astype(jnp.bfloat16)
            elif h < H + KVH:
                k_ref[0, h - H, :, :] = _rope(yh, cos, sin).astype(jnp.bfloat16)
            else:
                v_ref[0, h - H - KVH, :, :] = yh.astype(jnp.bfloat16)


def _flash_kernel(q_ref, k_ref, v_ref, wo_ref, mask_ref, o_ref):
    qi = pl.program_id(1)

    m0 = jnp.full((MQ, 1), -1e30, jnp.bfloat16)
    l0 = jnp.zeros((MQ, 1), jnp.float32)
    acc0 = jnp.zeros((MQ, D), jnp.float32)

    out = jnp.zeros((BQ, HID), jnp.float32)
    for gp in range(KVH // 2):
        ga, gb = 2 * gp, 2 * gp + 1
        qa = q_ref[0, GROUPS * ga:GROUPS * (ga + 1)].reshape(MQ, D)
        qb = q_ref[0, GROUPS * gb:GROUPS * (gb + 1)].reshape(MQ, D)

        def one(g, q, kb, m, l, acc, masked):
            ks = k_ref[0, g, pl.ds(kb * BK, BK), :]
            vs = v_ref[0, g, pl.ds(kb * BK, BK), :]
            s = jax.lax.dot_general(q, ks, (((1,), (1,)), ((), ())),
                                    preferred_element_type=jnp.float32
                                    ).astype(jnp.bfloat16)
            if masked:
                s = s + mask_ref[...]
            m_new = jnp.maximum(m, s.max(axis=1, keepdims=True))
            alpha = jax.lax.exp2((m - m_new).astype(jnp.float32))
            p = jax.lax.exp2(s - m_new)
            l_new = l * alpha + p.sum(axis=1, keepdims=True, dtype=jnp.float32)
            acc_new = acc * alpha + jnp.dot(p, vs,
                                            preferred_element_type=jnp.float32)
            return m_new, l_new, acc_new

        def block(kb, carry, masked):
            ma, la, acca, mb, lb, accb = carry
            ma, la, acca = one(ga, qa, kb, ma, la, acca, masked)
            mb, lb, accb = one(gb, qb, kb, mb, lb, accb, masked)
            return ma, la, acca, mb, lb, accb

        carry = jax.lax.fori_loop(0, qi, lambda kb, c: block(kb, c, False),
                                  (m0, l0, acc0, m0, l0, acc0))
        ma, la, acca, mb, lb, accb = block(qi, carry, True)

        for g, l, acc in ((ga, la, acca), (gb, lb, accb)):
            attn = (acc * (1.0 / l)).astype(jnp.bfloat16)
            attn_w = jnp.concatenate(
                [attn[j * BQ:(j + 1) * BQ, :] for j in range(GROUPS)], axis=1)
            out = out + jnp.dot(
                attn_w, wo_ref[GROUPS * D * g:GROUPS * D * (g + 1), :],
                preferred_element_type=jnp.float32)

    o_ref[0] = out


def kernel(hidden_states, Wq, Wk, Wv, Wo):
    inv_freq = 1.0 / (THETA ** (jnp.arange(0, D, 2, dtype=jnp.float32) / D))
    t = jnp.arange(S, dtype=jnp.float32)
    freqs = jnp.outer(t, inv_freq)
    emb = jnp.concatenate([freqs, freqs], axis=-1)
    cos = jnp.cos(emb)
    sin = jnp.sin(emb)
    wqkv = jnp.concatenate([Wq * SCALE, Wk, Wv], axis=1).astype(jnp.bfloat16)
    r = jnp.arange(MQ, dtype=jnp.int32) % BQ
    c = jnp.arange(BK, dtype=jnp.int32)
    mask_add = jnp.where(r[:, None] >= c[None, :], 0.0, -1e30
                         ).astype(jnp.bfloat16)
    wo16 = Wo.astype(jnp.bfloat16)
    x16 = hidden_states.astype(jnp.bfloat16)

    q, k, v = pl.pallas_call(
        _qkv_proj_kernel,
        grid=(B, S // BM),
        in_specs=[
            pl.BlockSpec((1, BM, HID), lambda b, m: (b, m, 0)),
            pl.BlockSpec((HID, (H + 2 * KVH) * D), lambda b, m: (0, 0)),
            pl.BlockSpec((BM, D), lambda b, m: (m, 0)),
            pl.BlockSpec((BM, D), lambda b, m: (m, 0)),
        ],
        out_specs=[
            pl.BlockSpec((1, H, BM, D), lambda b, m: (b, 0, m, 0)),
            pl.BlockSpec((1, KVH, BM, D), lambda b, m: (b, 0, m, 0)),
            pl.BlockSpec((1, KVH, BM, D), lambda b, m: (b, 0, m, 0)),
        ],
        out_shape=[
            jax.ShapeDtypeStruct((B, H, S, D), jnp.bfloat16),
            jax.ShapeDtypeStruct((B, KVH, S, D), jnp.bfloat16),
            jax.ShapeDtypeStruct((B, KVH, S, D), jnp.bfloat16),
        ],
        compiler_params=pltpu.CompilerParams(
            dimension_semantics=("parallel", "arbitrary")),
    )(x16, wqkv, cos, sin)

    out = pl.pallas_call(
        _flash_kernel,
        grid=(B, S // BQ),
        in_specs=[
            pl.BlockSpec((1, H, BQ, D), lambda b, i: (b, 0, i, 0)),
            pl.BlockSpec((1, KVH, S, D), lambda b, i: (b, 0, 0, 0)),
            pl.BlockSpec((1, KVH, S, D), lambda b, i: (b, 0, 0, 0)),
            pl.BlockSpec((H * D, HID), lambda b, i: (0, 0)),
            pl.BlockSpec((MQ, BK), lambda b, i: (0, 0)),
        ],
        out_specs=pl.BlockSpec((1, BQ, HID), lambda b, i: (b, i, 0)),
        out_shape=jax.ShapeDtypeStruct((B, S, HID), jnp.float32),
        compiler_params=pltpu.CompilerParams(
            dimension_semantics=("parallel", "arbitrary")),
    )(q, k, v, wo16, mask_add)

    return out


# speedup vs baseline: 1.0185x; 1.0013x over previous
import jax
import jax.numpy as jnp
from jax.experimental import pallas as pl
from jax.experimental.pallas import tpu as pltpu

B, S, HID = 2, 2048, 2048
H, KVH, D = 16, 4, 128
THETA = 10000.0
GROUPS = H // KVH

BM = 512
BQ = 512
BK = 512
MQ = GROUPS * BQ
SCALE = 1.4426950408889634 / (D ** 0.5)


def _rope(x, cos, sin):
    rot = jnp.concatenate([-x[:, D // 2:], x[:, : D // 2]], axis=1)
    return x * cos + rot * sin


def _qkv_proj_kernel(x_ref, w_ref, cos_ref, sin_ref, q_ref, k_ref, v_ref):
    x = x_ref[0]
    cos = cos_ref[...]
    sin = sin_ref[...]
    for c in range((H + 2 * KVH) // 4):
        y = jnp.dot(x, w_ref[:, 4 * c * D:4 * (c + 1) * D],
                    preferred_element_type=jnp.float32)
        for j in range(4):
            h = 4 * c + j
            yh = y[:, j * D:(j + 1) * D]
            if h < H:
                q_ref[0, h, :, :] = _rope(yh, cos, sin).astype(jnp.bfloat16)
            elif h < H + KVH:
                k_ref[0, h - H, :, :] = _rope(yh, cos, sin).astype(jnp.bfloat16)
            else:
                v_ref[0, h - H - KVH, :, :] = yh.astype(jnp.bfloat16)


def _flash_kernel(q_ref, k_ref, v_ref, wo_ref, mask_ref, o_ref):
    qi = pl.program_id(1)

    m0 = jnp.full((MQ, 1), -1e30, jnp.bfloat16)
    l0 = jnp.zeros((MQ, 1), jnp.float32)
    acc0 = jnp.zeros((MQ, D), jnp.float32)

    out = jnp.zeros((BQ, HID), jnp.float32)
    for gp in range(KVH // 2):
        ga, gb = 2 * gp, 2 * gp + 1
        qa = q_ref[0, GROUPS * ga:GROUPS * (ga + 1)].reshape(MQ, D)
        qb = q_ref[0, GROUPS * gb:GROUPS * (gb + 1)].reshape(MQ, D)

        def one(g, q, kb, m, l, acc, masked):
            ks = k_ref[0, g, pl.ds(kb * BK, BK), :]
            vs = v_ref[0, g, pl.ds(kb * BK, BK), :]
            s = jax.lax.dot_general(q, ks, (((1,), (1,)), ((), ())),
                                    preferred_element_type=jnp.float32
                                    ).astype(jnp.bfloat16)
            if masked:
                s = s + mask_ref[...]
            m_new = jnp.maximum(m, s.max(axis=1, keepdims=True))
            alpha = jax.lax.exp2((m - m_new).astype(jnp.float32))
            p = jax.lax.exp2(s - m_new)
            l_new = l * alpha + p.sum(axis=1, keepdims=True, dtype=jnp.float32)
            acc_new = acc * alpha + jnp.dot(p, vs,
                                            preferred_element_type=jnp.float32)
            return m_new, l_new, acc_new

        def block(kb, carry, masked):
            ma, la, acca, mb, lb, accb = carry
            ma, la, acca = one(ga, qa, kb, ma, la, acca, masked)
            mb, lb, accb = one(gb, qb, kb, mb, lb, accb, masked)
            return ma, la, acca, mb, lb, accb

        carry = jax.lax.fori_loop(0, qi, lambda kb, c: block(kb, c, False),
                                  (m0, l0, acc0, m0, l0, acc0))
        ma, la, acca, mb, lb, accb = block(qi, carry, True)

        for g, l, acc in ((ga, la, acca), (gb, lb, accb)):
            attn = (acc * (1.0 / l)).astype(jnp.bfloat16)
            attn_w = jnp.concatenate(
                [attn[j * BQ:(j + 1) * BQ, :] for j in range(GROUPS)], axis=1)
            out = out + jnp.dot(
                attn_w, wo_ref[GROUPS * D * g:GROUPS * D * (g + 1), :],
                preferred_element_type=jnp.float32)

    o_ref[0] = out


def kernel(hidden_states, Wq, Wk, Wv, Wo):
    inv_freq = 1.0 / (THETA ** (jnp.arange(0, D, 2, dtype=jnp.float32) / D))
    t = jnp.arange(S, dtype=jnp.float32)
    freqs = jnp.outer(t, inv_freq)
    emb = jnp.concatenate([freqs, freqs], axis=-1)
    cos = jnp.cos(emb)
    sin = jnp.sin(emb)
    wqkv = jnp.concatenate([Wq * SCALE, Wk, Wv], axis=1).astype(jnp.bfloat16)
    r = jnp.arange(MQ, dtype=jnp.int32) % BQ
    c = jnp.arange(BK, dtype=jnp.int32)
    mask_add = jnp.where(r[:, None] >= c[None, :], 0.0, -1e30
                         ).astype(jnp.bfloat16)
    wo16 = Wo.astype(jnp.bfloat16)
    x16 = hidden_states.astype(jnp.bfloat16)

    q, k, v = pl.pallas_call(
        _qkv_proj_kernel,
        grid=(B, S // BM),
        in_specs=[
            pl.BlockSpec((1, BM, HID), lambda b, m: (b, m, 0)),
            pl.BlockSpec((HID, (H + 2 * KVH) * D), lambda b, m: (0, 0)),
            pl.BlockSpec((BM, D), lambda b, m: (m, 0)),
            pl.BlockSpec((BM, D), lambda b, m: (m, 0)),
        ],
        out_specs=[
            pl.BlockSpec((1, H, BM, D), lambda b, m: (b, 0, m, 0)),
            pl.BlockSpec((1, KVH, BM, D), lambda b, m: (b, 0, m, 0)),
            pl.BlockSpec((1, KVH, BM, D), lambda b, m: (b, 0, m, 0)),
        ],
        out_shape=[
            jax.ShapeDtypeStruct((B, H, S, D), jnp.bfloat16),
            jax.ShapeDtypeStruct((B, KVH, S, D), jnp.bfloat16),
            jax.ShapeDtypeStruct((B, KVH, S, D), jnp.bfloat16),
        ],
        compiler_params=pltpu.CompilerParams(
            dimension_semantics=("parallel", "parallel")),
    )(x16, wqkv, cos, sin)

    out = pl.pallas_call(
        _flash_kernel,
        grid=(B, S // BQ),
        in_specs=[
            pl.BlockSpec((1, H, BQ, D), lambda b, i: (b, 0, i, 0)),
            pl.BlockSpec((1, KVH, S, D), lambda b, i: (b, 0, 0, 0)),
            pl.BlockSpec((1, KVH, S, D), lambda b, i: (b, 0, 0, 0)),
            pl.BlockSpec((H * D, HID), lambda b, i: (0, 0)),
            pl.BlockSpec((MQ, BK), lambda b, i: (0, 0)),
        ],
        out_specs=pl.BlockSpec((1, BQ, HID), lambda b, i: (b, i, 0)),
        out_shape=jax.ShapeDtypeStruct((B, S, HID), jnp.float32),
        compiler_params=pltpu.CompilerParams(
            dimension_semantics=("parallel", "parallel")),
    )(q, k, v, wo16, mask_add)

    return out
